# baseline (speedup 1.0000x reference)
import functools

import jax
import jax.numpy as jnp
from jax import lax
from jax.experimental import pallas as pl
from jax.experimental.pallas import tpu as pltpu
from jax.experimental.pallas import tpu_sc as plsc

BATCH = 16384
EMBEDDING_DIM = 64

_INFO = plsc.get_sparse_core_info()
_NC = _INFO.num_cores
_NS = _INFO.num_subcores
_NW = _NC * _NS
_B_PER_W = BATCH // _NW
_CHUNK = 128
_NCHUNK = _B_PER_W // _CHUNK


@functools.partial(
    pl.kernel,
    mesh=plsc.VectorSubcoreMesh(core_axis_name="c", subcore_axis_name="s"),
    out_type=jax.ShapeDtypeStruct((BATCH, EMBEDDING_DIM), jnp.float32),
    scratch_types=[
        pltpu.VMEM((_B_PER_W,), jnp.int32),
        pltpu.VMEM((_B_PER_W, EMBEDDING_DIM), jnp.float32),
        pltpu.SemaphoreType.DMA,
    ],
    compiler_params=pltpu.CompilerParams(use_tc_tiling_on_sc=False),
)
def _sc_gather(idx_hbm, table_hbm, out_hbm, idx_v, rows_v, sem):
    wid = lax.axis_index("s") * _NC + lax.axis_index("c")
    base = wid * _B_PER_W
    pltpu.sync_copy(idx_hbm.at[pl.ds(base, _B_PER_W)], idx_v)
    copies = [
        pltpu.async_copy(
            table_hbm.at[idx_v.at[pl.ds(j * _CHUNK, _CHUNK)]],
            rows_v.at[pl.ds(j * _CHUNK, _CHUNK)],
            sem,
        )
        for j in range(_NCHUNK)
    ]
    for c in copies:
        c.wait()
    pltpu.sync_copy(rows_v, out_hbm.at[pl.ds(base, _B_PER_W)])


def kernel(geohash_indices, embedding_table):
    idx = geohash_indices.astype(jnp.int32)
    return _sc_gather(idx, embedding_table)


# speedup vs baseline: 2.9812x; 2.9812x over previous
import functools

import jax
import jax.numpy as jnp
from jax import lax
from jax.experimental import pallas as pl
from jax.experimental.pallas import tpu as pltpu
from jax.experimental.pallas import tpu_sc as plsc

BATCH = 16384
EMBEDDING_DIM = 64
NUM_ROWS = 1_000_000

_INFO = plsc.get_sparse_core_info()
_NC = _INFO.num_cores
_NS = _INFO.num_subcores
_NW = _NC * _NS
_L = 16

_SLOT_W = 512
_NSLOT = (NUM_ROWS + _SLOT_W - 1) // _SLOT_W
_LAST_BASE = 999552
_NVEC = BATCH // _L
_HITS_CAP = BATCH + _L


@functools.partial(
    pl.kernel,
    mesh=plsc.VectorSubcoreMesh(core_axis_name="c", subcore_axis_name="s"),
    out_type=jax.ShapeDtypeStruct((BATCH * EMBEDDING_DIM,), jnp.float32),
    scratch_types=[
        pltpu.VMEM((BATCH,), jnp.int32),
        pltpu.VMEM((_HITS_CAP,), jnp.int32),
        pltpu.VMEM((_HITS_CAP,), jnp.int32),
        pltpu.VMEM((_HITS_CAP,), jnp.int32),
        pltpu.VMEM((_HITS_CAP,), jnp.int32),
        pltpu.VMEM((8, 4, 8, 128), jnp.float32),
        pltpu.VMEM((_L * EMBEDDING_DIM,), jnp.float32),
        pltpu.SemaphoreType.DMA,
        pltpu.SemaphoreType.DMA,
    ],
    compiler_params=pltpu.CompilerParams(
        use_tc_tiling_on_sc=True,
        disable_bounds_checks=True,
        needs_layout_passes=False,
    ),
)
def _sc_gather(idx_hbm, table_t_hbm, out_hbm, idx_v, hit_r, hit_b, slot_lr,
               slot_b, block_v, stage_v, sem_in, sem_out):
    wid = lax.axis_index("s") * _NC + lax.axis_index("c")
    s_lo = wid * _NSLOT // _NW
    s_hi = (wid + 1) * _NSLOT // _NW
    lo = s_lo * _SLOT_W
    hi = jnp.minimum(s_hi * _SLOT_W, NUM_ROWS)
    lanes = lax.iota(jnp.int32, _L)

    pltpu.sync_copy(idx_hbm, idx_v)

    def scan_strip(v, cnt):
        rv = idx_v[pl.ds(v * _L, _L)]
        m = (rv >= lo) & (rv < hi)
        mi = m.astype(jnp.int32)
        pos = cnt + plsc.cumsum(mi) - mi
        plsc.store_scatter(hit_r, [pos], rv, mask=m)
        plsc.store_scatter(hit_b, [pos], v * _L + lanes, mask=m)
        return cnt + plsc.all_reduce_population_count(m)[0]

    nhit = lax.fori_loop(0, _NVEC, scan_strip, jnp.int32(0))
    nhit_vecs = (nhit + _L - 1) // _L

    g_of_c = [(k * _L + lanes) // 8 for k in range(4)]
    cc_of_c = lanes % 8

    def do_slot(s, carry):
        base = s * _SLOT_W
        top = jnp.minimum(base + _SLOT_W, NUM_ROWS)
        rc0 = pl.multiple_of(jnp.minimum(base, _LAST_BASE), 128)

        loads = []
        for g in range(8):
            for t in range(4):
                loads.append(pltpu.async_copy(
                    table_t_hbm.at[pl.ds(g * 8, 8), pl.ds(rc0 + t * 128, 128)],
                    block_v.at[g, t],
                    sem_in,
                ))

        def scan_hits(u, cnt):
            rv = hit_r[pl.ds(u * _L, _L)]
            bv = hit_b[pl.ds(u * _L, _L)]
            m = (rv >= base) & (rv < top) & (u * _L + lanes < nhit)
            mi = m.astype(jnp.int32)
            pos = cnt + plsc.cumsum(mi) - mi
            plsc.store_scatter(slot_lr, [pos], rv - rc0, mask=m)
            plsc.store_scatter(slot_b, [pos], bv, mask=m)
            return cnt + plsc.all_reduce_population_count(m)[0]

        ns = lax.fori_loop(0, nhit_vecs, scan_hits, jnp.int32(0))

        for c in loads:
            c.wait()

        def hit_group(h, carry2):
            lrv = slot_lr[pl.ds(h * _L, _L)]
            bv = slot_b[pl.ds(h * _L, _L)]
            for j in range(_L):
                @pl.when(h * _L + j < ns)
                def _fire(j=j, lrv=lrv, bv=bv):
                    t = lrv[j] // 128
                    rr = lrv[j] % 128
                    tv = jnp.full((_L,), t, jnp.int32)
                    rv = jnp.full((_L,), rr, jnp.int32)
                    for k in range(4):
                        vals = plsc.load_gather(
                            block_v, [g_of_c[k], tv, cc_of_c, rv])
                        stage_v[pl.ds(j * EMBEDDING_DIM + k * _L, _L)] = vals
                    off = pl.multiple_of(bv[j] * EMBEDDING_DIM, 8)
                    pltpu.async_copy(
                        stage_v.at[pl.ds(j * EMBEDDING_DIM, EMBEDDING_DIM)],
                        out_hbm.at[pl.ds(off, EMBEDDING_DIM)],
                        sem_out,
                    )
            for j in range(_L):
                @pl.when(h * _L + j < ns)
                def _drain(j=j):
                    pltpu.make_async_copy(
                        out_hbm.at[pl.ds(0, EMBEDDING_DIM)],
                        stage_v.at[pl.ds(j * EMBEDDING_DIM, EMBEDDING_DIM)],
                        sem_out,
                    ).wait()
            return carry2

        lax.fori_loop(0, (ns + _L - 1) // _L, hit_group, jnp.int32(0))
        return carry

    lax.fori_loop(s_lo, s_hi, do_slot, jnp.int32(0))


def kernel(geohash_indices, embedding_table):
    idx = geohash_indices.astype(jnp.int32)
    out_flat = _sc_gather(idx, embedding_table.T)
    return out_flat.reshape(BATCH, EMBEDDING_DIM)


# speedup vs baseline: 3.0146x; 1.0112x over previous
import functools

import jax
import jax.numpy as jnp
from jax import lax
from jax.experimental import pallas as pl
from jax.experimental.pallas import tpu as pltpu
from jax.experimental.pallas import tpu_sc as plsc

BATCH = 16384
EMBEDDING_DIM = 64
NUM_ROWS = 1_000_000

_INFO = plsc.get_sparse_core_info()
_NC = _INFO.num_cores
_NS = _INFO.num_subcores
_NW = _NC * _NS
_L = 16

_SLOT_W = 512
_NSLOT = (NUM_ROWS + _SLOT_W - 1) // _SLOT_W
_LAST_BASE = 999552
_NVEC = BATCH // _L
_HITS_CAP = BATCH + _L


@functools.partial(
    pl.kernel,
    mesh=plsc.VectorSubcoreMesh(core_axis_name="c", subcore_axis_name="s"),
    out_type=jax.ShapeDtypeStruct((BATCH * EMBEDDING_DIM,), jnp.float32),
    scratch_types=[
        pltpu.VMEM((BATCH,), jnp.int32),
        pltpu.VMEM((_HITS_CAP,), jnp.int32),
        pltpu.VMEM((_HITS_CAP,), jnp.int32),
        pltpu.VMEM((_HITS_CAP,), jnp.int32),
        pltpu.VMEM((_HITS_CAP,), jnp.int32),
        pltpu.VMEM((EMBEDDING_DIM, _SLOT_W), jnp.float32),
        pltpu.VMEM((_L * EMBEDDING_DIM,), jnp.float32),
        pltpu.SemaphoreType.DMA,
        pltpu.SemaphoreType.DMA,
    ],
    compiler_params=pltpu.CompilerParams(
        use_tc_tiling_on_sc=True,
        disable_bounds_checks=True,
        needs_layout_passes=False,
    ),
)
def _sc_gather(idx_hbm, table_t_hbm, out_hbm, idx_v, hit_r, hit_b, slot_lr,
               slot_b, block_v, stage_v, sem_in, sem_out):
    wid = lax.axis_index("s") * _NC + lax.axis_index("c")
    s_lo = wid * _NSLOT // _NW
    s_hi = (wid + 1) * _NSLOT // _NW
    lo = s_lo * _SLOT_W
    hi = jnp.minimum(s_hi * _SLOT_W, NUM_ROWS)
    lanes = lax.iota(jnp.int32, _L)

    pltpu.sync_copy(idx_hbm, idx_v)

    def scan_strip(v, cnt):
        rv = idx_v[pl.ds(v * _L, _L)]
        m = (rv >= lo) & (rv < hi)
        mi = m.astype(jnp.int32)
        pos = cnt + plsc.cumsum(mi) - mi
        plsc.store_scatter(hit_r, [pos], rv, mask=m)
        plsc.store_scatter(hit_b, [pos], v * _L + lanes, mask=m)
        return cnt + plsc.all_reduce_population_count(m)[0]

    nhit = lax.fori_loop(0, _NVEC, scan_strip, jnp.int32(0))
    nhit_vecs = (nhit + _L - 1) // _L

    c_of_k = [k * _L + lanes for k in range(4)]

    def do_slot(s, carry):
        base = s * _SLOT_W
        top = jnp.minimum(base + _SLOT_W, NUM_ROWS)
        rc0 = pl.multiple_of(jnp.minimum(base, _LAST_BASE), 128)

        load = pltpu.async_copy(
            table_t_hbm.at[pl.ds(0, EMBEDDING_DIM), pl.ds(rc0, _SLOT_W)],
            block_v,
            sem_in,
        )

        def scan_hits(u, cnt):
            rv = hit_r[pl.ds(u * _L, _L)]
            bv = hit_b[pl.ds(u * _L, _L)]
            m = (rv >= base) & (rv < top) & (u * _L + lanes < nhit)
            mi = m.astype(jnp.int32)
            pos = cnt + plsc.cumsum(mi) - mi
            plsc.store_scatter(slot_lr, [pos], rv - rc0, mask=m)
            plsc.store_scatter(slot_b, [pos], bv, mask=m)
            return cnt + plsc.all_reduce_population_count(m)[0]

        ns = lax.fori_loop(0, nhit_vecs, scan_hits, jnp.int32(0))

        load.wait()

        def hit_group(h, carry2):
            lrv = slot_lr[pl.ds(h * _L, _L)]
            bv = slot_b[pl.ds(h * _L, _L)]
            for j in range(_L):
                @pl.when(h * _L + j < ns)
                def _fire(j=j, lrv=lrv, bv=bv):
                    cv = jnp.full((_L,), lrv[j], jnp.int32)
                    for k in range(4):
                        vals = plsc.load_gather(block_v, [c_of_k[k], cv])
                        stage_v[pl.ds(j * EMBEDDING_DIM + k * _L, _L)] = vals
                    off = pl.multiple_of(bv[j] * EMBEDDING_DIM, 8)
                    pltpu.async_copy(
                        stage_v.at[pl.ds(j * EMBEDDING_DIM, EMBEDDING_DIM)],
                        out_hbm.at[pl.ds(off, EMBEDDING_DIM)],
                        sem_out,
                    )
            for j in range(_L):
                @pl.when(h * _L + j < ns)
                def _drain(j=j):
                    pltpu.make_async_copy(
                        out_hbm.at[pl.ds(0, EMBEDDING_DIM)],
                        stage_v.at[pl.ds(j * EMBEDDING_DIM, EMBEDDING_DIM)],
                        sem_out,
                    ).wait()
            return carry2

        lax.fori_loop(0, (ns + _L - 1) // _L, hit_group, jnp.int32(0))
        return carry

    lax.fori_loop(s_lo, s_hi, do_slot, jnp.int32(0))


def kernel(geohash_indices, embedding_table):
    idx = geohash_indices.astype(jnp.int32)
    out_flat = _sc_gather(idx, embedding_table.T)
    return out_flat.reshape(BATCH, EMBEDDING_DIM)


# speedup vs baseline: 4.4344x; 1.4710x over previous
import functools

import jax
import jax.numpy as jnp
from jax import lax
from jax.experimental import pallas as pl
from jax.experimental.pallas import tpu as pltpu
from jax.experimental.pallas import tpu_sc as plsc

BATCH = 16384
EMBEDDING_DIM = 64
NUM_ROWS = 1_000_000

_INFO = plsc.get_sparse_core_info()
_NC = _INFO.num_cores
_NS = _INFO.num_subcores
_NW = _NC * _NS
_L = 16

_SLOT_W = 512
_NSLOT = (NUM_ROWS + _SLOT_W - 1) // _SLOT_W
_LAST_BASE = 999552
_NVEC = BATCH // _L
_HITS_CAP = BATCH + _L


@functools.partial(
    pl.kernel,
    mesh=plsc.VectorSubcoreMesh(core_axis_name="c", subcore_axis_name="s"),
    out_type=jax.ShapeDtypeStruct((BATCH * EMBEDDING_DIM,), jnp.float32),
    scratch_types=[
        pltpu.VMEM((BATCH,), jnp.int32),
        pltpu.VMEM((_HITS_CAP,), jnp.int32),
        pltpu.VMEM((_HITS_CAP,), jnp.int32),
        pltpu.VMEM((2, EMBEDDING_DIM, _SLOT_W), jnp.float32),
        pltpu.VMEM((_L * EMBEDDING_DIM,), jnp.float32),
        pltpu.SemaphoreType.DMA,
        pltpu.SemaphoreType.DMA,
        pltpu.SemaphoreType.DMA,
    ],
    compiler_params=pltpu.CompilerParams(
        use_tc_tiling_on_sc=True,
        disable_bounds_checks=True,
        needs_layout_passes=False,
    ),
)
def _sc_gather(idx_hbm, table_t_hbm, out_hbm, idx_v, hit_p, slot_p, block_v,
               stage_v, sem_b0, sem_b1, sem_out):
    wid = lax.axis_index("s") * _NC + lax.axis_index("c")
    s_lo = wid * _NSLOT // _NW
    s_hi = (wid + 1) * _NSLOT // _NW
    lo = s_lo * _SLOT_W
    hi = jnp.minimum(s_hi * _SLOT_W, NUM_ROWS)
    lanes = lax.iota(jnp.int32, _L)
    sems = [sem_b0, sem_b1]

    pltpu.sync_copy(idx_hbm, idx_v)

    def scan_strip(v, cnt):
        rv = idx_v[pl.ds(v * _L, _L)]
        m = (rv >= lo) & (rv < hi)
        mi = m.astype(jnp.int32)
        pos = cnt + plsc.cumsum(mi) - mi
        packed = (rv - lo) | ((v * _L + lanes) << 15)
        plsc.store_scatter(hit_p, [pos], packed, mask=m)
        return cnt + plsc.all_reduce_population_count(m)[0]

    nhit = lax.fori_loop(0, _NVEC, scan_strip, jnp.int32(0))
    nhit_vecs = (nhit + _L - 1) // _L

    c_of_k = [k * _L + lanes for k in range(4)]

    def rc0_of(s):
        return pl.multiple_of(jnp.minimum(s * _SLOT_W, _LAST_BASE), 128)

    def fire(s, p):
        pltpu.async_copy(
            table_t_hbm.at[pl.ds(0, EMBEDDING_DIM), pl.ds(rc0_of(s), _SLOT_W)],
            block_v.at[p],
            sems[p],
        )

    def wait_block(p):
        pltpu.make_async_copy(
            table_t_hbm.at[pl.ds(0, EMBEDDING_DIM), pl.ds(0, _SLOT_W)],
            block_v.at[p],
            sems[p],
        ).wait()

    def compress_slot(s):
        base = s * _SLOT_W
        top = jnp.minimum(base + _SLOT_W, NUM_ROWS)
        rc0 = rc0_of(s)

        def scan_hits(u, cnt):
            hp = hit_p[pl.ds(u * _L, _L)]
            rv = lo + (hp & 0x7FFF)
            m = (rv >= base) & (rv < top) & (u * _L + lanes < nhit)
            mi = m.astype(jnp.int32)
            pos = cnt + plsc.cumsum(mi) - mi
            packed = (rv - rc0) | ((hp >> 15) << 10)
            plsc.store_scatter(slot_p, [pos], packed, mask=m)
            return cnt + plsc.all_reduce_population_count(m)[0]

        return lax.fori_loop(0, nhit_vecs, scan_hits, jnp.int32(0))

    def process(blk, ns):
        def hit_group(h, carry2):
            spv = slot_p[pl.ds(h * _L, _L)]
            lrv = spv & 0x3FF
            bv = spv >> 10
            for j in range(_L):
                @pl.when(h * _L + j < ns)
                def _fire(j=j, lrv=lrv, bv=bv):
                    cv = jnp.full((_L,), lrv[j], jnp.int32)
                    for k in range(4):
                        vals = plsc.load_gather(blk, [c_of_k[k], cv])
                        stage_v[pl.ds(j * EMBEDDING_DIM + k * _L, _L)] = vals
                    off = pl.multiple_of(bv[j] * EMBEDDING_DIM, 8)
                    pltpu.async_copy(
                        stage_v.at[pl.ds(j * EMBEDDING_DIM, EMBEDDING_DIM)],
                        out_hbm.at[pl.ds(off, EMBEDDING_DIM)],
                        sem_out,
                    )
            for j in range(_L):
                @pl.when(h * _L + j < ns)
                def _drain(j=j):
                    pltpu.make_async_copy(
                        out_hbm.at[pl.ds(0, EMBEDDING_DIM)],
                        stage_v.at[pl.ds(j * EMBEDDING_DIM, EMBEDDING_DIM)],
                        sem_out,
                    ).wait()
            return carry2

        lax.fori_loop(0, (ns + _L - 1) // _L, hit_group, jnp.int32(0))

    @pl.when(s_lo < s_hi)
    def _prime():
        fire(s_lo, 0)

    def do_slot(i, carry):
        par = (i - s_lo) % 2

        @pl.when(i + 1 < s_hi)
        def _next():
            @pl.when(par == 0)
            def _():
                fire(i + 1, 1)

            @pl.when(par == 1)
            def _():
                fire(i + 1, 0)

        ns = compress_slot(i)

        @pl.when(par == 0)
        def _p0():
            wait_block(0)
            process(block_v.at[0], ns)

        @pl.when(par == 1)
        def _p1():
            wait_block(1)
            process(block_v.at[1], ns)

        return carry

    lax.fori_loop(s_lo, s_hi, do_slot, jnp.int32(0))


def kernel(geohash_indices, embedding_table):
    idx = geohash_indices.astype(jnp.int32)
    out_flat = _sc_gather(idx, embedding_table.T)
    return out_flat.reshape(BATCH, EMBEDDING_DIM)
